# baseline (speedup 1.0000x reference)
import functools

import jax
import jax.numpy as jnp
from jax import lax
from jax.experimental import pallas as pl
from jax.experimental.pallas import tpu as pltpu
from jax.experimental.pallas import tpu_sc as plsc

B = 16384
D = 32
NLAB = 5
LANES = 16
NC = 2
NS = 16
NW = NC * NS
BPW = B // NW
DIV = 8
WIN = BPW // DIV
PAD = BPW - WIN
NGROUP = BPW // LANES

_MESH = plsc.VectorSubcoreMesh(core_axis_name="c", subcore_axis_name="s")


def _body(uid_hbm, iid_hbm, uemb_hbm, iemb_hbm, ibias_hbm, ubeta_hbm,
          out_hbm, uid_v, iid_v, ue_v, ie_v, ib_v, ub_v, out_v, sem):
    wid = lax.axis_index("s") * NC + lax.axis_index("c")

    pltpu.sync_copy(uid_hbm.at[wid], uid_v)
    pltpu.sync_copy(iid_hbm.at[wid], iid_v)

    copies = []
    for k in range(DIV):
        isl = pl.ds(k * WIN, BPW)
        copies.append(pltpu.async_copy(
            uemb_hbm.at[uid_v.at[isl]], ue_v.at[isl], sem))
        copies.append(pltpu.async_copy(
            iemb_hbm.at[iid_v.at[isl]], ie_v.at[isl], sem))
        copies.append(pltpu.async_copy(
            ibias_hbm.at[iid_v.at[isl]], ib_v.at[isl], sem))
        copies.append(pltpu.async_copy(
            ubeta_hbm.at[uid_v.at[isl]], ub_v.at[isl], sem))
    for c in copies:
        c.wait()

    iota16 = lax.iota(jnp.int32, LANES)

    def group(g, carry):
        rows = g * LANES + iota16
        acc = plsc.load_gather(ib_v, [rows, jnp.zeros((LANES,), jnp.int32)])
        for d in range(D):
            cols = jnp.full((LANES,), d, jnp.int32)
            u = plsc.load_gather(ue_v, [rows, cols])
            it = plsc.load_gather(ie_v, [rows, cols])
            acc = acc + u * it
        cum = plsc.load_gather(ub_v, [rows, jnp.zeros((LANES,), jnp.int32)])
        uds = [1.0 / (1.0 + jnp.exp(acc - cum))]
        for j in range(1, NLAB - 1):
            bj = plsc.load_gather(ub_v, [rows, jnp.full((LANES,), j, jnp.int32)])
            cum = cum + jnp.exp(bj)
            uds.append(1.0 / (1.0 + jnp.exp(acc - cum)))
        outs = [uds[0]]
        for j in range(1, NLAB - 1):
            outs.append(uds[j] - uds[j - 1])
        outs.append(1.0 - uds[NLAB - 2])
        for j in range(NLAB):
            plsc.store_scatter(out_v, [rows, jnp.full((LANES,), j, jnp.int32)],
                               outs[j])
        return carry

    lax.fori_loop(0, NGROUP, group, 0)
    pltpu.sync_copy(out_v, out_hbm.at[wid])


@functools.partial(jax.jit, static_argnums=())
def kernel(user_ids, item_ids, user_emb, item_emb, item_bias, user_betas):
    pad = jnp.zeros((NW, PAD), jnp.int32)
    uid_r = jnp.concatenate([user_ids.reshape(NW, BPW), pad], axis=1)
    iid_r = jnp.concatenate([item_ids.reshape(NW, BPW), pad], axis=1)

    k = pl.kernel(
        _body,
        out_type=jax.ShapeDtypeStruct((NW, BPW, NLAB), jnp.float32),
        mesh=_MESH,
        compiler_params=pltpu.CompilerParams(
            needs_layout_passes=False, use_tc_tiling_on_sc=False),
        scratch_types=[
            pltpu.VMEM((BPW + PAD,), jnp.int32),
            pltpu.VMEM((BPW + PAD,), jnp.int32),
            pltpu.VMEM((BPW + PAD, D), jnp.float32),
            pltpu.VMEM((BPW + PAD, D), jnp.float32),
            pltpu.VMEM((BPW + PAD, 1), jnp.float32),
            pltpu.VMEM((BPW + PAD, NLAB - 1), jnp.float32),
            pltpu.VMEM((BPW, NLAB), jnp.float32),
            pltpu.SemaphoreType.DMA,
        ],
    )
    out = k(uid_r, iid_r, user_emb, item_emb, item_bias, user_betas)
    return out.reshape(B, NLAB)


# speedup vs baseline: 1.0002x; 1.0002x over previous
import functools

import jax
import jax.numpy as jnp
from jax import lax
from jax.experimental import pallas as pl
from jax.experimental.pallas import tpu as pltpu
from jax.experimental.pallas import tpu_sc as plsc

B = 16384
D = 32
NLAB = 5
LANES = 16
NC = 2
NS = 16
NW = NC * NS
BPW = B // NW
DIV = 8
WIN = BPW // DIV
PAD = BPW - WIN
NGROUP = BPW // LANES

_MESH = plsc.VectorSubcoreMesh(core_axis_name="c", subcore_axis_name="s")


def _body(uid_hbm, iid_hbm, uemb_hbm, iemb_hbm, ibias_hbm, ubeta_hbm,
          out_hbm, uid_v, iid_v, ue_v, ie_v, ib_v, ub_v, out_v, sem):
    wid = lax.axis_index("s") * NC + lax.axis_index("c")

    pltpu.sync_copy(uid_hbm.at[wid], uid_v)
    pltpu.sync_copy(iid_hbm.at[wid], iid_v)

    copies = []
    for k in range(DIV):
        isl = pl.ds(k * WIN, BPW)
        copies.append(pltpu.async_copy(
            uemb_hbm.at[uid_v.at[isl]], ue_v.at[isl], sem))
        copies.append(pltpu.async_copy(
            iemb_hbm.at[iid_v.at[isl]], ie_v.at[isl], sem))
        copies.append(pltpu.async_copy(
            ibias_hbm.at[iid_v.at[isl]], ib_v.at[isl], sem))
        copies.append(pltpu.async_copy(
            ubeta_hbm.at[uid_v.at[isl]], ub_v.at[isl], sem))
    for c in copies:
        c.wait()

    iota16 = lax.iota(jnp.int32, LANES)

    def group(g, carry):
        rows = g * LANES + iota16
        acc = plsc.load_gather(ib_v, [rows, jnp.zeros((LANES,), jnp.int32)])
        for d in range(D):
            cols = jnp.full((LANES,), d, jnp.int32)
            u = plsc.load_gather(ue_v, [rows, cols])
            it = plsc.load_gather(ie_v, [rows, cols])
            acc = acc + u * it
        cum = plsc.load_gather(ub_v, [rows, jnp.zeros((LANES,), jnp.int32)])
        uds = [1.0 / (1.0 + jnp.exp(acc - cum))]
        for j in range(1, NLAB - 1):
            bj = plsc.load_gather(ub_v, [rows, jnp.full((LANES,), j, jnp.int32)])
            cum = cum + jnp.exp(bj)
            uds.append(1.0 / (1.0 + jnp.exp(acc - cum)))
        outs = [uds[0]]
        for j in range(1, NLAB - 1):
            outs.append(uds[j] - uds[j - 1])
        outs.append(1.0 - uds[NLAB - 2])
        for j in range(NLAB):
            plsc.store_scatter(out_v, [rows, jnp.full((LANES,), j, jnp.int32)],
                               outs[j])
        return carry

    lax.fori_loop(0, NGROUP, group, 0)
    pltpu.sync_copy(out_v, out_hbm.at[wid])


@functools.partial(jax.jit, static_argnums=())
def kernel(user_ids, item_ids, user_emb, item_emb, item_bias, user_betas):
    pad = jnp.zeros((NW, PAD), jnp.int32)
    uid_r = jnp.concatenate([user_ids.reshape(NW, BPW), pad], axis=1)
    iid_r = jnp.concatenate([item_ids.reshape(NW, BPW), pad], axis=1)

    k = pl.kernel(
        _body,
        out_type=jax.ShapeDtypeStruct((NW, BPW, NLAB), jnp.float32),
        mesh=_MESH,
        compiler_params=pltpu.CompilerParams(
            needs_layout_passes=False, use_tc_tiling_on_sc=False,
            skip_device_barrier=True),
        scratch_types=[
            pltpu.VMEM((BPW + PAD,), jnp.int32),
            pltpu.VMEM((BPW + PAD,), jnp.int32),
            pltpu.VMEM((BPW + PAD, D), jnp.float32),
            pltpu.VMEM((BPW + PAD, D), jnp.float32),
            pltpu.VMEM((BPW + PAD, 1), jnp.float32),
            pltpu.VMEM((BPW + PAD, NLAB - 1), jnp.float32),
            pltpu.VMEM((BPW, NLAB), jnp.float32),
            pltpu.SemaphoreType.DMA,
        ],
    )
    out = k(uid_r, iid_r, user_emb, item_emb, item_bias, user_betas)
    return out.reshape(B, NLAB)
